# baseline (speedup 1.0000x reference)
import jax
import jax.numpy as jnp
from jax import lax
from jax.experimental import pallas as pl
from jax.experimental.pallas import tpu as pltpu
from jax.experimental.pallas import tpu_sc as plsc

_N = 10000
_E = 320000
_V = 1024
_F = 128
_NW = 32
_EW = _E // _NW
_NP = 10240
_CPW = _NP // _NW
_EC = 1024
_EP = 327680
_EWP = _EP // _NW

_MESH = plsc.VectorSubcoreMesh(core_axis_name="c", subcore_axis_name="s")
_SC_PARAMS = pltpu.CompilerParams(needs_layout_passes=False)

_Z16F = None


def _elu(v):
    return jnp.where(v > 0, v, jnp.exp(jnp.minimum(v, 0.0)) - 1.0)


def _dense1_body(x_ref, wemb_ref, bemb_ref, wg_ref, as_ref, ad_ref,
                 h_ref, ls_ref, ld_ref):
    emb = jnp.dot(x_ref[...], wemb_ref[...],
                  preferred_element_type=jnp.float32) + bemb_ref[...]
    h = jnp.dot(emb, wg_ref[...], preferred_element_type=jnp.float32)
    h_ref[...] = h
    hb = h.astype(jnp.bfloat16).astype(jnp.float32)
    asb = as_ref[...].astype(jnp.bfloat16).astype(jnp.float32)
    adb = ad_ref[...].astype(jnp.bfloat16).astype(jnp.float32)
    ls_ref[...] = jnp.sum(hb * asb, axis=1, keepdims=True)
    ld_ref[...] = jnp.sum(hb * adb, axis=1, keepdims=True)


def _dense1(x, W_emb, b_emb, Wg, a_s, a_d):
    n, f = x.shape
    nh = Wg.shape[1]
    blk = 2000
    return pl.pallas_call(
        _dense1_body,
        grid=(n // blk,),
        in_specs=[
            pl.BlockSpec((blk, f), lambda i: (i, 0)),
            pl.BlockSpec((f, nh), lambda i: (0, 0)),
            pl.BlockSpec((nh,), lambda i: (0,)),
            pl.BlockSpec((nh, nh), lambda i: (0, 0)),
            pl.BlockSpec((1, nh), lambda i: (0, 0)),
            pl.BlockSpec((1, nh), lambda i: (0, 0)),
        ],
        out_specs=[
            pl.BlockSpec((blk, nh), lambda i: (i, 0)),
            pl.BlockSpec((blk, 1), lambda i: (i, 0)),
            pl.BlockSpec((blk, 1), lambda i: (i, 0)),
        ],
        out_shape=[
            jax.ShapeDtypeStruct((n, nh), jnp.float32),
            jax.ShapeDtypeStruct((n, 1), jnp.float32),
            jax.ShapeDtypeStruct((n, 1), jnp.float32),
        ],
    )(x, W_emb, b_emb, Wg, a_s.reshape(1, -1), a_d.reshape(1, -1))


def _dense2_body(p0_ref, p1_ref, wg_ref, as_ref, ad_ref,
                 h_ref, ls_ref, ld_ref):
    h1 = _elu(p0_ref[...] + p1_ref[...])
    h = jnp.dot(h1, wg_ref[...], preferred_element_type=jnp.float32)
    h_ref[...] = h
    hb = h.astype(jnp.bfloat16).astype(jnp.float32)
    asb = as_ref[...].astype(jnp.bfloat16).astype(jnp.float32)
    adb = ad_ref[...].astype(jnp.bfloat16).astype(jnp.float32)
    ls_ref[...] = jnp.sum(hb * asb, axis=1, keepdims=True)
    ld_ref[...] = jnp.sum(hb * adb, axis=1, keepdims=True)


def _dense2(p0, p1, Wg, a_s, a_d):
    n, f = p0.shape
    nh = Wg.shape[1]
    blk = 2000
    return pl.pallas_call(
        _dense2_body,
        grid=(n // blk,),
        in_specs=[
            pl.BlockSpec((blk, f), lambda i: (i, 0)),
            pl.BlockSpec((blk, f), lambda i: (i, 0)),
            pl.BlockSpec((f, nh), lambda i: (0, 0)),
            pl.BlockSpec((1, nh), lambda i: (0, 0)),
            pl.BlockSpec((1, nh), lambda i: (0, 0)),
        ],
        out_specs=[
            pl.BlockSpec((blk, nh), lambda i: (i, 0)),
            pl.BlockSpec((blk, 1), lambda i: (i, 0)),
            pl.BlockSpec((blk, 1), lambda i: (i, 0)),
        ],
        out_shape=[
            jax.ShapeDtypeStruct((n, nh), jnp.float32),
            jax.ShapeDtypeStruct((n, 1), jnp.float32),
            jax.ShapeDtypeStruct((n, 1), jnp.float32),
        ],
    )(p0, p1, Wg, a_s.reshape(1, -1), a_d.reshape(1, -1))


def _support_body(p0_ref, p1_ref, wc_ref, sup_ref):
    embv = _elu(p0_ref[...] + p1_ref[...])
    sup_ref[...] = jnp.dot(embv, wc_ref[...],
                           preferred_element_type=jnp.float32)


def _support_tc(p0, p1, Wc):
    return pl.pallas_call(
        _support_body,
        out_shape=jax.ShapeDtypeStruct((_V, Wc.shape[1]), jnp.float32),
    )(p0, p1, Wc)


def _finalize_body(sup_ref, aggp_ref, degp_ref, bc_ref, out_ref):
    a = aggp_ref[...]
    agg2 = jnp.sum(a, axis=0)
    agg = jnp.concatenate([agg2[0], agg2[1]], axis=-1)
    deg = jnp.sum(degp_ref[...], axis=0)
    sup = sup_ref[...]
    out_ref[...] = (agg + sup) / (deg[:, None] + 1.0) + bc_ref[...]


def _finalize_tc(sup, aggp, degp, bc):
    return pl.pallas_call(
        _finalize_body,
        out_shape=jax.ShapeDtypeStruct((_V, sup.shape[1]), jnp.float32),
    )(sup, aggp, degp, bc)


def _wid():
    return lax.axis_index("s") * 2 + lax.axis_index("c")


def _e1_body(ls_hbm, ld_hbm, src_hbm, dst_hbm, e_hbm, mx_hbm,
             ls_v, ld_v, src_v, dst_v, e_v, mx_v):
    w = _wid()
    base = w * _EWP
    pltpu.sync_copy(ls_hbm, ls_v)
    pltpu.sync_copy(ld_hbm, ld_v)
    pltpu.sync_copy(src_hbm.at[pl.ds(base, _EWP)], src_v)
    pltpu.sync_copy(dst_hbm.at[pl.ds(base, _EWP)], dst_v)

    def step(i, mx):
        sl = pl.ds(i * 16, 16)
        s = plsc.load_gather(ls_v, [src_v[sl]])
        d = plsc.load_gather(ld_v, [dst_v[sl]])
        lg = s + d
        e = jnp.where(lg > 0, lg, 0.2 * lg)
        e_v[sl] = e
        return jnp.maximum(mx, e)

    mx = lax.fori_loop(0, _EWP // 16, step,
                       jnp.full((16,), -1e30, jnp.float32))
    mx_v[...] = mx
    pltpu.sync_copy(mx_v, mx_hbm.at[pl.ds(w * 16, 16)])
    pltpu.sync_copy(e_v, e_hbm.at[pl.ds(base, _EWP)])


def _sc_e1(ls, ld, src, dst):
    return pl.kernel(
        _e1_body,
        out_type=(jax.ShapeDtypeStruct((_EP,), jnp.float32),
                  jax.ShapeDtypeStruct((_NW * 16,), jnp.float32)),
        mesh=_MESH,
        compiler_params=_SC_PARAMS,
        scratch_types=[
            pltpu.VMEM((_NP,), jnp.float32),
            pltpu.VMEM((_NP,), jnp.float32),
            pltpu.VMEM((_EWP,), jnp.int32),
            pltpu.VMEM((_EWP,), jnp.int32),
            pltpu.VMEM((_EWP,), jnp.float32),
            pltpu.VMEM((16,), jnp.float32),
        ],
    )(ls, ld, src, dst)


def _e2_body(e_hbm, dst_hbm, k_hbm, ex_hbm, denp_hbm,
             e_v, dst_v, den_v, k_v):
    w = _wid()
    base = w * _EWP
    pltpu.sync_copy(e_hbm.at[pl.ds(base, _EWP)], e_v)
    pltpu.sync_copy(dst_hbm.at[pl.ds(base, _EWP)], dst_v)
    pltpu.sync_copy(k_hbm, k_v)
    kv = k_v[...]
    zero = jnp.zeros((16,), jnp.float32)

    def zstep(i, carry):
        den_v[pl.ds(i * 16, 16)] = zero
        return carry

    lax.fori_loop(0, _NP // 16, zstep, 0)

    def step(i, carry):
        sl = pl.ds(i * 16, 16)
        exv = jnp.exp(e_v[sl] - kv)
        e_v[sl] = exv
        plsc.addupdate_scatter(den_v, [dst_v[sl]], exv)
        return carry

    lax.fori_loop(0, _EWP // 16, step, 0)
    pltpu.sync_copy(e_v, ex_hbm.at[pl.ds(base, _EWP)])
    pltpu.sync_copy(den_v, denp_hbm.at[pl.ds(w * _NP, _NP)])


def _sc_e2(e, dst, kb):
    return pl.kernel(
        _e2_body,
        out_type=(jax.ShapeDtypeStruct((_EP,), jnp.float32),
                  jax.ShapeDtypeStruct((_NW * _NP,), jnp.float32)),
        mesh=_MESH,
        compiler_params=_SC_PARAMS,
        scratch_types=[
            pltpu.VMEM((_EWP,), jnp.float32),
            pltpu.VMEM((_EWP,), jnp.int32),
            pltpu.VMEM((_NP,), jnp.float32),
            pltpu.VMEM((16,), jnp.float32),
        ],
    )(e, dst, kb)


def _e3_body(denp_hbm, rden_hbm, row_v, acc_v):
    w = _wid()
    c0 = w * _CPW
    zero = jnp.zeros((16,), jnp.float32)
    for v in range(_CPW // 16):
        acc_v[pl.ds(v * 16, 16)] = zero

    def rstep(r, carry):
        pltpu.sync_copy(denp_hbm.at[pl.ds(r * _NP + c0, _CPW)], row_v)
        for v in range(_CPW // 16):
            sl = pl.ds(v * 16, 16)
            acc_v[sl] = acc_v[sl] + row_v[sl]
        return carry

    lax.fori_loop(0, _NW, rstep, 0)
    for v in range(_CPW // 16):
        sl = pl.ds(v * 16, 16)
        a = acc_v[sl]
        acc_v[sl] = jnp.where(a > 0, 1.0 / jnp.maximum(a, 1e-38), 0.0)
    pltpu.sync_copy(acc_v, rden_hbm.at[pl.ds(c0, _CPW)])


def _sc_e3(denp):
    return pl.kernel(
        _e3_body,
        out_type=jax.ShapeDtypeStruct((_NP,), jnp.float32),
        mesh=_MESH,
        compiler_params=_SC_PARAMS,
        scratch_types=[
            pltpu.VMEM((_CPW,), jnp.float32),
            pltpu.VMEM((_CPW,), jnp.float32),
        ],
    )(denp)


def _e4_body(h_hbm, ex_hbm, rden_hbm, src_hbm, dst_hbm,
             outp_hbm, rden_v, ex_v, src_v, dst_v, rows_v, shared, *sems):
    gsem = sems[:4]
    ssem = sems[4:]
    cid = lax.axis_index("c")
    sid = lax.axis_index("s")
    w = sid * 2 + cid
    base = w * _EWP
    pltpu.sync_copy(rden_hbm, rden_v)

    zero = jnp.zeros((16,), jnp.float32)
    for j in range(16):
        for c in range(8):
            rows_v[j, pl.ds(c * 16, 16)] = zero

    def zstep(i, carry):
        pltpu.sync_copy(rows_v.at[pl.ds(0, 16)],
                        shared.at[pl.ds(sid * 640 + i * 16, 16)])
        return carry

    lax.fori_loop(0, 40, zstep, 0)
    plsc.subcore_barrier()

    def _fire_gather(b, s):
        pltpu.async_copy(h_hbm.at[src_v.at[pl.ds(s * 64, 64)]],
                         rows_v.at[pl.ds(b * 64, 64)], gsem[b])

    def _wait_gather(b):
        pltpu.make_async_copy(h_hbm.at[src_v.at[pl.ds(0, 64)]],
                              rows_v.at[pl.ds(b * 64, 64)], gsem[b]).wait()

    def _fire_scatter(b, s):
        for q in range(4):
            didx = dst_v[pl.ds(s * 64 + q * 16, 16)]
            pltpu.async_copy(rows_v.at[pl.ds(b * 64 + q * 16, 16)],
                             shared.at[didx], ssem[b], add=True)

    def _drain_scatter(b):
        for q in range(4):
            pltpu.make_async_copy(rows_v.at[pl.ds(b * 64 + q * 16, 16)],
                                  shared.at[dst_v[pl.ds(0, 16)]],
                                  ssem[b]).wait()

    def _compute(b, s):
        for q in range(4):
            sl = pl.ds(s * 64 + q * 16, 16)
            att = ex_v[sl] * plsc.load_gather(rden_v, [dst_v[sl]])
            for j in range(16):
                a = att[j]
                r = b * 64 + q * 16 + j
                for c in range(8):
                    s2 = (r, pl.ds(c * 16, 16))
                    rows_v[s2] = rows_v[s2] * a

    def _step(b, s, prefetch):
        if prefetch:
            bp = (b + 2) % 4

            @pl.when(s >= 2)
            def _():
                _drain_scatter(bp)

            _fire_gather(bp, s + 2)
        _wait_gather(b)
        _compute(b, s)
        _fire_scatter(b, s)

    def chunk(ch, carry):
        cbase = base + ch * _EC
        pltpu.sync_copy(ex_hbm.at[pl.ds(cbase, _EC)], ex_v)
        pltpu.sync_copy(src_hbm.at[pl.ds(cbase, _EC)], src_v)
        pltpu.sync_copy(dst_hbm.at[pl.ds(cbase, _EC)], dst_v)
        _fire_gather(0, 0)
        _fire_gather(1, 1)

        def block(it, carry2):
            s0 = it * 4
            for b in range(4):
                _step(b, s0 + b, prefetch=True)
            return carry2

        lax.fori_loop(0, 3, block, 0)
        for s in range(12, 14):
            b = s % 4
            _drain_scatter((b + 2) % 4)
            _fire_gather((b + 2) % 4, s + 2)
            _step(b, s, prefetch=False)
        for s in range(14, 16):
            _step(s % 4, s, prefetch=False)
        for b in range(4):
            _drain_scatter(b)
        return carry

    lax.fori_loop(0, _EWP // _EC, chunk, 0)
    plsc.subcore_barrier()
    sl = pl.ds(sid * 640, 640)
    pltpu.sync_copy(shared.at[sl], outp_hbm.at[cid, sl])


def _sc_e4(h, ex, rden, src, dst):
    return pl.kernel(
        _e4_body,
        out_type=jax.ShapeDtypeStruct((2, _NP, _F), jnp.float32),
        mesh=_MESH,
        compiler_params=_SC_PARAMS,
        scratch_types=[
            pltpu.VMEM((_NP,), jnp.float32),
            pltpu.VMEM((_EC,), jnp.float32),
            pltpu.VMEM((_EC,), jnp.int32),
            pltpu.VMEM((_EC,), jnp.int32),
            pltpu.VMEM((256, _F), jnp.float32),
            pltpu.VMEM_SHARED((_NP, _F), jnp.float32),
        ] + [pltpu.SemaphoreType.DMA] * 8,
    )(h, ex, rden, src, dst)


_ECH = _E // 16
_ESUB = 4000


def _f1_body(sup_hbm, src_hbm, dst_hbm, aggp_hbm, degp_hbm,
             sup_v, src_v, dst_v, agg_v, deg_v):
    cid = lax.axis_index("c")
    sid = lax.axis_index("s")
    ebase = sid * _ECH
    c0 = cid * 32
    pltpu.sync_copy(sup_hbm, sup_v)
    zero = jnp.zeros((16,), jnp.float32)

    def z1(i, carry):
        agg_v[pl.ds(i * 16, 16)] = zero
        return carry

    lax.fori_loop(0, (_V * 32) // 16, z1, 0)

    def z2(i, carry):
        deg_v[pl.ds(i * 16, 16)] = zero
        return carry

    lax.fori_loop(0, _V // 16, z2, 0)

    def sub(s, carry):
        pltpu.sync_copy(src_hbm.at[pl.ds(ebase + s * _ESUB, _ESUB)], src_v)
        pltpu.sync_copy(dst_hbm.at[pl.ds(ebase + s * _ESUB, _ESUB)], dst_v)

        def step(i, carry2):
            sl = pl.ds(i * 16, 16)
            sidx = src_v[sl]
            didx = dst_v[sl]
            mask = (sidx < _V) & (didx < _V)
            nhit = plsc.all_reduce_population_count(mask)

            @pl.when(nhit[0] > 0)
            def _():
                sidx_s = jnp.where(mask, sidx, 0)
                didx_s = jnp.where(mask, didx, 0)
                sflat = sidx_s * 64 + c0
                dflat = didx_s * 32
                for c in range(32):
                    v = plsc.load_gather(sup_v, [sflat + c])
                    plsc.addupdate_scatter(agg_v, [dflat + c], v, mask=mask)
                plsc.addupdate_scatter(
                    deg_v, [didx_s], jnp.where(mask, 1.0, 0.0))

            return carry2

        lax.fori_loop(0, _ESUB // 16, step, 0)
        return carry

    lax.fori_loop(0, _ECH // _ESUB, sub, 0)
    w = sid * 2 + cid
    pltpu.sync_copy(agg_v, aggp_hbm.at[pl.ds(w * _V * 32, _V * 32)])

    @pl.when(cid == 0)
    def _():
        pltpu.sync_copy(deg_v, degp_hbm.at[pl.ds(sid * _V, _V)])


def _sc_f1(sup_flat, src, dst):
    return pl.kernel(
        _f1_body,
        out_type=(jax.ShapeDtypeStruct((_NW * _V * 32,), jnp.float32),
                  jax.ShapeDtypeStruct((16 * _V,), jnp.float32)),
        mesh=_MESH,
        compiler_params=_SC_PARAMS,
        scratch_types=[
            pltpu.VMEM((_V * 64,), jnp.float32),
            pltpu.VMEM((_ESUB,), jnp.int32),
            pltpu.VMEM((_ESUB,), jnp.int32),
            pltpu.VMEM((_V * 32,), jnp.float32),
            pltpu.VMEM((_V,), jnp.float32),
        ],
    )(sup_flat, src, dst)


def _gat_layer_sc(h, ls, ld, src, dst):
    e, mx = _sc_e1(ls, ld, src, dst)
    kb = jnp.broadcast_to(jnp.max(mx), (16,))
    ex, denp = _sc_e2(e, dst, kb)
    rden = _sc_e3(denp)
    return _sc_e4(h, ex, rden, src, dst)


def kernel(x, adj, W_emb, b_emb, Wg1, a1s, a1d, Wg2, a2s, a2d, Wc, bc):
    src = adj[0].astype(jnp.int32)
    dst = adj[1].astype(jnp.int32)
    npad = _EP - _E
    srcp = jnp.concatenate([src, jnp.zeros((npad,), jnp.int32)])
    dstp = jnp.concatenate([dst, jnp.full((npad,), _NP - 1, jnp.int32)])
    zpadn = jnp.zeros((_NP - _N,), jnp.float32)

    h1p, ls1, ld1 = _dense1(x, W_emb, b_emb, Wg1, a1s, a1d)
    p1 = _gat_layer_sc(h1p, jnp.concatenate([ls1[:, 0], zpadn]),
                       jnp.concatenate([ld1[:, 0], zpadn]), srcp, dstp)

    h2p, ls2, ld2 = _dense2(p1[0, :_N], p1[1, :_N], Wg2, a2s, a2d)
    p2 = _gat_layer_sc(h2p, jnp.concatenate([ls2[:, 0], zpadn]),
                       jnp.concatenate([ld2[:, 0], zpadn]), srcp, dstp)

    sup = _support_tc(p2[0, :_V], p2[1, :_V], Wc)
    aggp, degp = _sc_f1(sup.reshape(-1), src, dst)
    return _finalize_tc(sup, aggp.reshape(16, 2, _V, 32),
                        degp.reshape(16, _V), bc)


# speedup vs baseline: 1.8651x; 1.8651x over previous
import jax
import jax.numpy as jnp
from jax import lax
from jax.experimental import pallas as pl
from jax.experimental.pallas import tpu as pltpu
from jax.experimental.pallas import tpu_sc as plsc

_N = 10000
_E = 320000
_V = 1024
_F = 128
_NW = 32
_EW = _E // _NW
_NP = 10240
_CPW = _NP // _NW
_EC = 2000

_MESH = plsc.VectorSubcoreMesh(core_axis_name="c", subcore_axis_name="s")
_SC_PARAMS = pltpu.CompilerParams(needs_layout_passes=False)

_Z16F = None


def _elu(v):
    return jnp.where(v > 0, v, jnp.exp(jnp.minimum(v, 0.0)) - 1.0)


def _dense1_body(x_ref, wemb_ref, bemb_ref, wg_ref, as_ref, ad_ref,
                 h_ref, ls_ref, ld_ref):
    emb = jnp.dot(x_ref[...], wemb_ref[...],
                  preferred_element_type=jnp.float32) + bemb_ref[...]
    h = jnp.dot(emb, wg_ref[...], preferred_element_type=jnp.float32)
    h_ref[...] = h
    hb = h.astype(jnp.bfloat16).astype(jnp.float32)
    asb = as_ref[...].astype(jnp.bfloat16).astype(jnp.float32)
    adb = ad_ref[...].astype(jnp.bfloat16).astype(jnp.float32)
    ls_ref[...] = jnp.sum(hb * asb, axis=1, keepdims=True)
    ld_ref[...] = jnp.sum(hb * adb, axis=1, keepdims=True)


def _dense1(x, W_emb, b_emb, Wg, a_s, a_d):
    n, f = x.shape
    nh = Wg.shape[1]
    blk = 2000
    return pl.pallas_call(
        _dense1_body,
        grid=(n // blk,),
        in_specs=[
            pl.BlockSpec((blk, f), lambda i: (i, 0)),
            pl.BlockSpec((f, nh), lambda i: (0, 0)),
            pl.BlockSpec((nh,), lambda i: (0,)),
            pl.BlockSpec((nh, nh), lambda i: (0, 0)),
            pl.BlockSpec((1, nh), lambda i: (0, 0)),
            pl.BlockSpec((1, nh), lambda i: (0, 0)),
        ],
        out_specs=[
            pl.BlockSpec((blk, nh), lambda i: (i, 0)),
            pl.BlockSpec((blk, 1), lambda i: (i, 0)),
            pl.BlockSpec((blk, 1), lambda i: (i, 0)),
        ],
        out_shape=[
            jax.ShapeDtypeStruct((n, nh), jnp.float32),
            jax.ShapeDtypeStruct((n, 1), jnp.float32),
            jax.ShapeDtypeStruct((n, 1), jnp.float32),
        ],
    )(x, W_emb, b_emb, Wg, a_s.reshape(1, -1), a_d.reshape(1, -1))


def _dense2_body(p0_ref, p1_ref, wg_ref, as_ref, ad_ref,
                 h_ref, ls_ref, ld_ref):
    h1 = _elu(p0_ref[...] + p1_ref[...])
    h = jnp.dot(h1, wg_ref[...], preferred_element_type=jnp.float32)
    h_ref[...] = h
    hb = h.astype(jnp.bfloat16).astype(jnp.float32)
    asb = as_ref[...].astype(jnp.bfloat16).astype(jnp.float32)
    adb = ad_ref[...].astype(jnp.bfloat16).astype(jnp.float32)
    ls_ref[...] = jnp.sum(hb * asb, axis=1, keepdims=True)
    ld_ref[...] = jnp.sum(hb * adb, axis=1, keepdims=True)


def _dense2(p0, p1, Wg, a_s, a_d):
    n, f = p0.shape
    nh = Wg.shape[1]
    blk = 2000
    return pl.pallas_call(
        _dense2_body,
        grid=(n // blk,),
        in_specs=[
            pl.BlockSpec((blk, f), lambda i: (i, 0)),
            pl.BlockSpec((blk, f), lambda i: (i, 0)),
            pl.BlockSpec((f, nh), lambda i: (0, 0)),
            pl.BlockSpec((1, nh), lambda i: (0, 0)),
            pl.BlockSpec((1, nh), lambda i: (0, 0)),
        ],
        out_specs=[
            pl.BlockSpec((blk, nh), lambda i: (i, 0)),
            pl.BlockSpec((blk, 1), lambda i: (i, 0)),
            pl.BlockSpec((blk, 1), lambda i: (i, 0)),
        ],
        out_shape=[
            jax.ShapeDtypeStruct((n, nh), jnp.float32),
            jax.ShapeDtypeStruct((n, 1), jnp.float32),
            jax.ShapeDtypeStruct((n, 1), jnp.float32),
        ],
    )(p0, p1, Wg, a_s.reshape(1, -1), a_d.reshape(1, -1))


def _support_body(p0_ref, p1_ref, wc_ref, sup_ref):
    embv = _elu(p0_ref[...] + p1_ref[...])
    sup_ref[...] = jnp.dot(embv, wc_ref[...],
                           preferred_element_type=jnp.float32)


def _support_tc(p0, p1, Wc):
    return pl.pallas_call(
        _support_body,
        out_shape=jax.ShapeDtypeStruct((_V, Wc.shape[1]), jnp.float32),
    )(p0, p1, Wc)


def _finalize_body(sup_ref, aggp_ref, degp_ref, bc_ref, out_ref):
    a = aggp_ref[...]
    agg2 = jnp.sum(a, axis=0)
    agg = jnp.concatenate([agg2[0], agg2[1]], axis=-1)
    deg = jnp.sum(degp_ref[...], axis=0)
    sup = sup_ref[...]
    out_ref[...] = (agg + sup) / (deg[:, None] + 1.0) + bc_ref[...]


def _finalize_tc(sup, aggp, degp, bc):
    return pl.pallas_call(
        _finalize_body,
        out_shape=jax.ShapeDtypeStruct((_V, sup.shape[1]), jnp.float32),
    )(sup, aggp, degp, bc)


def _wid():
    return lax.axis_index("s") * 2 + lax.axis_index("c")


def _e1_body(ls_hbm, ld_hbm, src_hbm, dst_hbm, e_hbm, mx_hbm,
             ls_v, ld_v, src_v, dst_v, e_v, mx_v):
    w = _wid()
    base = w * _EW
    pltpu.sync_copy(ls_hbm, ls_v)
    pltpu.sync_copy(ld_hbm, ld_v)
    pltpu.sync_copy(src_hbm.at[pl.ds(base, _EW)], src_v)
    pltpu.sync_copy(dst_hbm.at[pl.ds(base, _EW)], dst_v)

    def step(i, mx):
        sl = pl.ds(i * 16, 16)
        s = plsc.load_gather(ls_v, [src_v[sl]])
        d = plsc.load_gather(ld_v, [dst_v[sl]])
        lg = s + d
        e = jnp.where(lg > 0, lg, 0.2 * lg)
        e_v[sl] = e
        return jnp.maximum(mx, e)

    mx = lax.fori_loop(0, _EW // 16, step,
                       jnp.full((16,), -1e30, jnp.float32))
    mx_v[...] = mx
    pltpu.sync_copy(mx_v, mx_hbm.at[pl.ds(w * 16, 16)])
    pltpu.sync_copy(e_v, e_hbm.at[pl.ds(base, _EW)])


def _sc_e1(ls, ld, src, dst):
    return pl.kernel(
        _e1_body,
        out_type=(jax.ShapeDtypeStruct((_E,), jnp.float32),
                  jax.ShapeDtypeStruct((_NW * 16,), jnp.float32)),
        mesh=_MESH,
        compiler_params=_SC_PARAMS,
        scratch_types=[
            pltpu.VMEM((_N,), jnp.float32),
            pltpu.VMEM((_N,), jnp.float32),
            pltpu.VMEM((_EW,), jnp.int32),
            pltpu.VMEM((_EW,), jnp.int32),
            pltpu.VMEM((_EW,), jnp.float32),
            pltpu.VMEM((16,), jnp.float32),
        ],
    )(ls, ld, src, dst)


def _e2_body(e_hbm, dst_hbm, mx_hbm, ex_hbm, denp_hbm,
             e_v, dst_v, den_v, k_v):
    w = _wid()
    base = w * _EW
    pltpu.sync_copy(e_hbm.at[pl.ds(base, _EW)], e_v)
    pltpu.sync_copy(dst_hbm.at[pl.ds(base, _EW)], dst_v)
    pltpu.sync_copy(mx_hbm, k_v)
    km = k_v[pl.ds(0, 16)]
    for i in range(1, _NW):
        km = jnp.maximum(km, k_v[pl.ds(i * 16, 16)])
    kv = jnp.broadcast_to(lax.reduce_max(km, (0,)), (16,))
    zero = jnp.zeros((16,), jnp.float32)

    def zstep(i, carry):
        den_v[pl.ds(i * 16, 16)] = zero
        return carry

    lax.fori_loop(0, _NP // 16, zstep, 0)

    def step(i, carry):
        sl = pl.ds(i * 16, 16)
        exv = jnp.exp(e_v[sl] - kv)
        e_v[sl] = exv
        plsc.addupdate_scatter(den_v, [dst_v[sl]], exv)
        return carry

    lax.fori_loop(0, _EW // 16, step, 0)
    pltpu.sync_copy(e_v, ex_hbm.at[pl.ds(base, _EW)])
    pltpu.sync_copy(den_v, denp_hbm.at[pl.ds(w * _NP, _NP)])


def _sc_e2(e, dst, mx):
    return pl.kernel(
        _e2_body,
        out_type=(jax.ShapeDtypeStruct((_E,), jnp.float32),
                  jax.ShapeDtypeStruct((_NW * _NP,), jnp.float32)),
        mesh=_MESH,
        compiler_params=_SC_PARAMS,
        scratch_types=[
            pltpu.VMEM((_EW,), jnp.float32),
            pltpu.VMEM((_EW,), jnp.int32),
            pltpu.VMEM((_NP,), jnp.float32),
            pltpu.VMEM((_NW * 16,), jnp.float32),
        ],
    )(e, dst, mx)


def _e3_body(denp_hbm, rden_hbm, row_v, acc_v):
    w = _wid()
    c0 = w * _CPW
    zero = jnp.zeros((16,), jnp.float32)
    for v in range(_CPW // 16):
        acc_v[pl.ds(v * 16, 16)] = zero

    def rstep(r, carry):
        pltpu.sync_copy(denp_hbm.at[pl.ds(r * _NP + c0, _CPW)], row_v)
        for v in range(_CPW // 16):
            sl = pl.ds(v * 16, 16)
            acc_v[sl] = acc_v[sl] + row_v[sl]
        return carry

    lax.fori_loop(0, _NW, rstep, 0)
    for v in range(_CPW // 16):
        sl = pl.ds(v * 16, 16)
        a = acc_v[sl]
        acc_v[sl] = jnp.where(a > 0, 1.0 / jnp.maximum(a, 1e-38), 0.0)
    pltpu.sync_copy(acc_v, rden_hbm.at[pl.ds(c0, _CPW)])


def _sc_e3(denp):
    return pl.kernel(
        _e3_body,
        out_type=jax.ShapeDtypeStruct((_NP,), jnp.float32),
        mesh=_MESH,
        compiler_params=_SC_PARAMS,
        scratch_types=[
            pltpu.VMEM((_CPW,), jnp.float32),
            pltpu.VMEM((_CPW,), jnp.float32),
        ],
    )(denp)


def _e4_body(restrict_v, h_hbm, ex_hbm, rden_hbm, src_hbm, dst_hbm,
             outp_hbm, rden_v, ex_v, src_v, dst_v, rows_v, shared, *sems):
    gsem = sems[:10]
    ssem = sems[10:]
    cid = lax.axis_index("c")
    sid = lax.axis_index("s")
    w = sid * 2 + cid
    base = w * _EW
    pltpu.sync_copy(rden_hbm, rden_v)

    zero = jnp.zeros((16,), jnp.float32)
    for j in range(16):
        for c in range(8):
            rows_v[j, pl.ds(c * 16, 16)] = zero

    def zstep(i, carry):
        pltpu.sync_copy(rows_v.at[pl.ds(0, 16)],
                        shared.at[pl.ds(sid * 640 + i * 16, 16)])
        return carry

    lax.fori_loop(0, 40, zstep, 0)
    plsc.subcore_barrier()

    def _didx(g):
        return dst_v[pl.ds(g * 16, 16)]

    def _fire_gather(b, g):
        pltpu.async_copy(h_hbm.at[src_v[pl.ds(g * 16, 16)]],
                         rows_v.at[pl.ds(b * 16, 16)], gsem[b])

    def _wait_gather(b):
        pltpu.make_async_copy(h_hbm.at[src_v[pl.ds(0, 16)]],
                              rows_v.at[pl.ds(b * 16, 16)], gsem[b]).wait()

    def _fire_scatter(b, g):
        pltpu.async_copy(rows_v.at[pl.ds(b * 16, 16)], shared.at[_didx(g)],
                         ssem[b], add=True)

    def _wait_scatter(b):
        pltpu.make_async_copy(rows_v.at[pl.ds(b * 16, 16)],
                              shared.at[_didx(0)], ssem[b]).wait()

    def _scale(b, g):
        att = ex_v[pl.ds(g * 16, 16)] * plsc.load_gather(rden_v, [_didx(g)])
        for j in range(16):
            a = att[j]
            r = b * 16 + j
            for c in range(8):
                s2 = (r, pl.ds(c * 16, 16))
                rows_v[s2] = rows_v[s2] * a

    def _compute(b, g):
        if restrict_v:
            nh = plsc.all_reduce_population_count(_didx(g) < _V)

            @pl.when(nh[0] > 0)
            def _():
                _scale(b, g)
        else:
            _scale(b, g)

    def chunk(ch, carry):
        cbase = base + ch * _EC
        pltpu.sync_copy(ex_hbm.at[pl.ds(cbase, _EC)], ex_v)
        pltpu.sync_copy(src_hbm.at[pl.ds(cbase, _EC)], src_v)
        pltpu.sync_copy(dst_hbm.at[pl.ds(cbase, _EC)], dst_v)
        for b in range(5):
            _fire_gather(b, b)

        def block(outer, carry2):
            g0 = outer * 10
            for b in range(10):
                g = g0 + b
                bp = (b + 5) % 10

                @pl.when(g >= 5)
                def _():
                    _wait_scatter(bp)

                _fire_gather(bp, g + 5)
                _wait_gather(b)
                _compute(b, g)
                _fire_scatter(b, g)
            return carry2

        lax.fori_loop(0, (_EC // 16 - 5) // 10, block, 0)
        for b in range(5):
            g = _EC // 16 - 5 + b
            _wait_gather(b)
            _compute(b, g)
            _fire_scatter(b, g)
        for b in range(10):
            _wait_scatter(b)
        return carry

    lax.fori_loop(0, _EW // _EC, chunk, 0)
    plsc.subcore_barrier()
    sl = pl.ds(sid * 640, 640)
    pltpu.sync_copy(shared.at[sl], outp_hbm.at[cid, sl])


def _sc_e4(h, ex, rden, src, dst, restrict_v=False):
    import functools as _ft
    return pl.kernel(
        _ft.partial(_e4_body, restrict_v),
        out_type=jax.ShapeDtypeStruct((2, _NP, _F), jnp.float32),
        mesh=_MESH,
        compiler_params=_SC_PARAMS,
        scratch_types=[
            pltpu.VMEM((_NP,), jnp.float32),
            pltpu.VMEM((_EC,), jnp.float32),
            pltpu.VMEM((_EC,), jnp.int32),
            pltpu.VMEM((_EC,), jnp.int32),
            pltpu.VMEM((160, _F), jnp.float32),
            pltpu.VMEM_SHARED((_NP, _F), jnp.float32),
        ] + [pltpu.SemaphoreType.DMA] * 20,
    )(h, ex, rden, src, dst)


_ECH = _E // 16
_ESUB = 4000


def _f1_body(sup_hbm, src_hbm, dst_hbm, aggp_hbm, degp_hbm,
             sup_v, src_v, dst_v, agg_v, deg_v, sup_sh):
    cid = lax.axis_index("c")
    sid = lax.axis_index("s")
    ebase = sid * _ECH
    c0 = cid * 32

    @pl.when(sid == 0)
    def _():
        pltpu.sync_copy(sup_hbm, sup_sh)

    plsc.subcore_barrier()
    pltpu.sync_copy(sup_sh, sup_v)
    zero = jnp.zeros((16,), jnp.float32)

    def z1(i, carry):
        agg_v[pl.ds(i * 16, 16)] = zero
        return carry

    lax.fori_loop(0, (_V * 32) // 16, z1, 0)

    def z2(i, carry):
        deg_v[pl.ds(i * 16, 16)] = zero
        return carry

    lax.fori_loop(0, _V // 16, z2, 0)

    def sub(s, carry):
        pltpu.sync_copy(src_hbm.at[pl.ds(ebase + s * _ESUB, _ESUB)], src_v)
        pltpu.sync_copy(dst_hbm.at[pl.ds(ebase + s * _ESUB, _ESUB)], dst_v)

        def step(i, carry2):
            sl = pl.ds(i * 16, 16)
            sidx = src_v[sl]
            didx = dst_v[sl]
            mask = (sidx < _V) & (didx < _V)
            nhit = plsc.all_reduce_population_count(mask)

            @pl.when(nhit[0] > 0)
            def _():
                sidx_s = jnp.where(mask, sidx, 0)
                didx_s = jnp.where(mask, didx, 0)
                sflat = sidx_s * 64 + c0
                dflat = didx_s * 32
                for c in range(32):
                    v = plsc.load_gather(sup_v, [sflat + c])
                    plsc.addupdate_scatter(agg_v, [dflat + c], v, mask=mask)
                plsc.addupdate_scatter(
                    deg_v, [didx_s], jnp.where(mask, 1.0, 0.0))

            return carry2

        lax.fori_loop(0, _ESUB // 16, step, 0)
        return carry

    lax.fori_loop(0, _ECH // _ESUB, sub, 0)
    w = sid * 2 + cid
    pltpu.sync_copy(agg_v, aggp_hbm.at[pl.ds(w * _V * 32, _V * 32)])

    @pl.when(cid == 0)
    def _():
        pltpu.sync_copy(deg_v, degp_hbm.at[pl.ds(sid * _V, _V)])


def _sc_f1(sup_flat, src, dst):
    return pl.kernel(
        _f1_body,
        out_type=(jax.ShapeDtypeStruct((_NW * _V * 32,), jnp.float32),
                  jax.ShapeDtypeStruct((16 * _V,), jnp.float32)),
        mesh=_MESH,
        compiler_params=_SC_PARAMS,
        scratch_types=[
            pltpu.VMEM((_V * 64,), jnp.float32),
            pltpu.VMEM((_ESUB,), jnp.int32),
            pltpu.VMEM((_ESUB,), jnp.int32),
            pltpu.VMEM((_V * 32,), jnp.float32),
            pltpu.VMEM((_V,), jnp.float32),
            pltpu.VMEM_SHARED((_V * 64,), jnp.float32),
        ],
    )(sup_flat, src, dst)


def _gat_layer_sc(h, ls, ld, src, dst, restrict_v=False):
    e, mx = _sc_e1(ls, ld, src, dst)
    ex, denp = _sc_e2(e, dst, mx)
    rden = _sc_e3(denp)
    return _sc_e4(h, ex, rden, src, dst, restrict_v)


def kernel(x, adj, W_emb, b_emb, Wg1, a1s, a1d, Wg2, a2s, a2d, Wc, bc):
    src = adj[0].astype(jnp.int32)
    dst = adj[1].astype(jnp.int32)

    h1p, ls1, ld1 = _dense1(x, W_emb, b_emb, Wg1, a1s, a1d)
    p1 = _gat_layer_sc(h1p, ls1[:, 0], ld1[:, 0], src, dst)

    h2p, ls2, ld2 = _dense2(p1[0, :_N], p1[1, :_N], Wg2, a2s, a2d)
    p2 = _gat_layer_sc(h2p, ls2[:, 0], ld2[:, 0], src, dst)

    sup = _support_tc(p2[0, :_V], p2[1, :_V], Wc)
    aggp, degp = _sc_f1(sup.reshape(-1), src, dst)
    return _finalize_tc(sup, aggp.reshape(16, 2, _V, 32),
                        degp.reshape(16, _V), bc)


# speedup vs baseline: 1.8689x; 1.0020x over previous
import jax
import jax.numpy as jnp
from jax import lax
from jax.experimental import pallas as pl
from jax.experimental.pallas import tpu as pltpu
from jax.experimental.pallas import tpu_sc as plsc

_N = 10000
_E = 320000
_V = 1024
_F = 128
_NW = 32
_EW = _E // _NW
_NP = 10240
_CPW = _NP // _NW
_EC = 2000

_MESH = plsc.VectorSubcoreMesh(core_axis_name="c", subcore_axis_name="s")
_SC_PARAMS = pltpu.CompilerParams(needs_layout_passes=False)

_Z16F = None


def _elu(v):
    return jnp.where(v > 0, v, jnp.exp(jnp.minimum(v, 0.0)) - 1.0)


def _dense1_body(x_ref, wemb_ref, bemb_ref, wg_ref, as_ref, ad_ref,
                 h_ref, ls_ref, ld_ref):
    emb = jnp.dot(x_ref[...], wemb_ref[...],
                  preferred_element_type=jnp.float32) + bemb_ref[...]
    h = jnp.dot(emb, wg_ref[...], preferred_element_type=jnp.float32)
    h_ref[...] = h
    hb = h.astype(jnp.bfloat16).astype(jnp.float32)
    asb = as_ref[...].astype(jnp.bfloat16).astype(jnp.float32)
    adb = ad_ref[...].astype(jnp.bfloat16).astype(jnp.float32)
    ls_ref[...] = jnp.sum(hb * asb, axis=1, keepdims=True)
    ld_ref[...] = jnp.sum(hb * adb, axis=1, keepdims=True)


def _dense1(x, W_emb, b_emb, Wg, a_s, a_d):
    n, f = x.shape
    nh = Wg.shape[1]
    blk = 2000
    return pl.pallas_call(
        _dense1_body,
        grid=(n // blk,),
        in_specs=[
            pl.BlockSpec((blk, f), lambda i: (i, 0)),
            pl.BlockSpec((f, nh), lambda i: (0, 0)),
            pl.BlockSpec((nh,), lambda i: (0,)),
            pl.BlockSpec((nh, nh), lambda i: (0, 0)),
            pl.BlockSpec((1, nh), lambda i: (0, 0)),
            pl.BlockSpec((1, nh), lambda i: (0, 0)),
        ],
        out_specs=[
            pl.BlockSpec((blk, nh), lambda i: (i, 0)),
            pl.BlockSpec((blk, 1), lambda i: (i, 0)),
            pl.BlockSpec((blk, 1), lambda i: (i, 0)),
        ],
        out_shape=[
            jax.ShapeDtypeStruct((n, nh), jnp.float32),
            jax.ShapeDtypeStruct((n, 1), jnp.float32),
            jax.ShapeDtypeStruct((n, 1), jnp.float32),
        ],
    )(x, W_emb, b_emb, Wg, a_s.reshape(1, -1), a_d.reshape(1, -1))


def _dense2_body(p0_ref, p1_ref, wg_ref, as_ref, ad_ref,
                 h_ref, ls_ref, ld_ref):
    h1 = _elu(p0_ref[...] + p1_ref[...])
    h = jnp.dot(h1, wg_ref[...], preferred_element_type=jnp.float32)
    h_ref[...] = h
    hb = h.astype(jnp.bfloat16).astype(jnp.float32)
    asb = as_ref[...].astype(jnp.bfloat16).astype(jnp.float32)
    adb = ad_ref[...].astype(jnp.bfloat16).astype(jnp.float32)
    ls_ref[...] = jnp.sum(hb * asb, axis=1, keepdims=True)
    ld_ref[...] = jnp.sum(hb * adb, axis=1, keepdims=True)


def _dense2(p0, p1, Wg, a_s, a_d):
    n, f = p0.shape
    nh = Wg.shape[1]
    blk = 2000
    return pl.pallas_call(
        _dense2_body,
        grid=(n // blk,),
        in_specs=[
            pl.BlockSpec((blk, f), lambda i: (i, 0)),
            pl.BlockSpec((blk, f), lambda i: (i, 0)),
            pl.BlockSpec((f, nh), lambda i: (0, 0)),
            pl.BlockSpec((1, nh), lambda i: (0, 0)),
            pl.BlockSpec((1, nh), lambda i: (0, 0)),
        ],
        out_specs=[
            pl.BlockSpec((blk, nh), lambda i: (i, 0)),
            pl.BlockSpec((blk, 1), lambda i: (i, 0)),
            pl.BlockSpec((blk, 1), lambda i: (i, 0)),
        ],
        out_shape=[
            jax.ShapeDtypeStruct((n, nh), jnp.float32),
            jax.ShapeDtypeStruct((n, 1), jnp.float32),
            jax.ShapeDtypeStruct((n, 1), jnp.float32),
        ],
    )(p0, p1, Wg, a_s.reshape(1, -1), a_d.reshape(1, -1))


def _support_body(p0_ref, p1_ref, wc_ref, sup_ref):
    embv = _elu(p0_ref[...] + p1_ref[...])
    sup_ref[...] = jnp.dot(embv, wc_ref[...],
                           preferred_element_type=jnp.float32)


def _support_tc(p0, p1, Wc):
    return pl.pallas_call(
        _support_body,
        out_shape=jax.ShapeDtypeStruct((_V, Wc.shape[1]), jnp.float32),
    )(p0, p1, Wc)


def _finalize_body(sup_ref, aggp_ref, degp_ref, bc_ref, out_ref):
    a = aggp_ref[...]
    agg2 = jnp.sum(a, axis=0)
    agg = jnp.concatenate([agg2[0], agg2[1]], axis=-1)
    deg = jnp.sum(degp_ref[...], axis=0)
    sup = sup_ref[...]
    out_ref[...] = (agg + sup) / (deg[:, None] + 1.0) + bc_ref[...]


def _finalize_tc(sup, aggp, degp, bc):
    return pl.pallas_call(
        _finalize_body,
        out_shape=jax.ShapeDtypeStruct((_V, sup.shape[1]), jnp.float32),
    )(sup, aggp, degp, bc)


def _wid():
    return lax.axis_index("s") * 2 + lax.axis_index("c")


def _e1_body(ls_hbm, ld_hbm, src_hbm, dst_hbm, e_hbm, mx_hbm,
             ls_v, ld_v, src_v, dst_v, e_v, mx_v):
    w = _wid()
    base = w * _EW
    pltpu.sync_copy(ls_hbm, ls_v)
    pltpu.sync_copy(ld_hbm, ld_v)
    pltpu.sync_copy(src_hbm.at[pl.ds(base, _EW)], src_v)
    pltpu.sync_copy(dst_hbm.at[pl.ds(base, _EW)], dst_v)

    def step(i, mx):
        sl = pl.ds(i * 16, 16)
        s = plsc.load_gather(ls_v, [src_v[sl]])
        d = plsc.load_gather(ld_v, [dst_v[sl]])
        lg = s + d
        e = jnp.where(lg > 0, lg, 0.2 * lg)
        e_v[sl] = e
        return jnp.maximum(mx, e)

    mx = lax.fori_loop(0, _EW // 16, step,
                       jnp.full((16,), -1e30, jnp.float32))
    mx_v[...] = mx
    pltpu.sync_copy(mx_v, mx_hbm.at[pl.ds(w * 16, 16)])
    pltpu.sync_copy(e_v, e_hbm.at[pl.ds(base, _EW)])


def _sc_e1(ls, ld, src, dst):
    return pl.kernel(
        _e1_body,
        out_type=(jax.ShapeDtypeStruct((_E,), jnp.float32),
                  jax.ShapeDtypeStruct((_NW * 16,), jnp.float32)),
        mesh=_MESH,
        compiler_params=_SC_PARAMS,
        scratch_types=[
            pltpu.VMEM((_N,), jnp.float32),
            pltpu.VMEM((_N,), jnp.float32),
            pltpu.VMEM((_EW,), jnp.int32),
            pltpu.VMEM((_EW,), jnp.int32),
            pltpu.VMEM((_EW,), jnp.float32),
            pltpu.VMEM((16,), jnp.float32),
        ],
    )(ls, ld, src, dst)


def _e2_body(e_hbm, dst_hbm, mx_hbm, ex_hbm, denp_hbm,
             e_v, dst_v, den_v, k_v):
    w = _wid()
    base = w * _EW
    pltpu.sync_copy(e_hbm.at[pl.ds(base, _EW)], e_v)
    pltpu.sync_copy(dst_hbm.at[pl.ds(base, _EW)], dst_v)
    pltpu.sync_copy(mx_hbm, k_v)
    km = k_v[pl.ds(0, 16)]
    for i in range(1, _NW):
        km = jnp.maximum(km, k_v[pl.ds(i * 16, 16)])
    kv = jnp.broadcast_to(lax.reduce_max(km, (0,)), (16,))
    zero = jnp.zeros((16,), jnp.float32)

    def zstep(i, carry):
        den_v[pl.ds(i * 16, 16)] = zero
        return carry

    lax.fori_loop(0, _NP // 16, zstep, 0)

    def step(i, carry):
        sl = pl.ds(i * 16, 16)
        exv = jnp.exp(e_v[sl] - kv)
        e_v[sl] = exv
        plsc.addupdate_scatter(den_v, [dst_v[sl]], exv)
        return carry

    lax.fori_loop(0, _EW // 16, step, 0)
    pltpu.sync_copy(e_v, ex_hbm.at[pl.ds(base, _EW)])
    pltpu.sync_copy(den_v, denp_hbm.at[pl.ds(w * _NP, _NP)])


def _sc_e2(e, dst, mx):
    return pl.kernel(
        _e2_body,
        out_type=(jax.ShapeDtypeStruct((_E,), jnp.float32),
                  jax.ShapeDtypeStruct((_NW * _NP,), jnp.float32)),
        mesh=_MESH,
        compiler_params=_SC_PARAMS,
        scratch_types=[
            pltpu.VMEM((_EW,), jnp.float32),
            pltpu.VMEM((_EW,), jnp.int32),
            pltpu.VMEM((_NP,), jnp.float32),
            pltpu.VMEM((_NW * 16,), jnp.float32),
        ],
    )(e, dst, mx)


def _e3_body(denp_hbm, rden_hbm, row_v, acc_v):
    w = _wid()
    c0 = w * _CPW
    zero = jnp.zeros((16,), jnp.float32)
    for v in range(_CPW // 16):
        acc_v[pl.ds(v * 16, 16)] = zero

    def rstep(r, carry):
        pltpu.sync_copy(denp_hbm.at[pl.ds(r * _NP + c0, _CPW)], row_v)
        for v in range(_CPW // 16):
            sl = pl.ds(v * 16, 16)
            acc_v[sl] = acc_v[sl] + row_v[sl]
        return carry

    lax.fori_loop(0, _NW, rstep, 0)
    for v in range(_CPW // 16):
        sl = pl.ds(v * 16, 16)
        a = acc_v[sl]
        acc_v[sl] = jnp.where(a > 0, 1.0 / jnp.maximum(a, 1e-38), 0.0)
    pltpu.sync_copy(acc_v, rden_hbm.at[pl.ds(c0, _CPW)])


def _sc_e3(denp):
    return pl.kernel(
        _e3_body,
        out_type=jax.ShapeDtypeStruct((_NP,), jnp.float32),
        mesh=_MESH,
        compiler_params=_SC_PARAMS,
        scratch_types=[
            pltpu.VMEM((_CPW,), jnp.float32),
            pltpu.VMEM((_CPW,), jnp.float32),
        ],
    )(denp)


def _e4_body(h_hbm, ex_hbm, rden_hbm, src_hbm, dst_hbm,
             outp_hbm, rden_v, ex_v, src_v, dst_v, rows_v, shared, *sems):
    gsem = sems[:10]
    ssem = sems[10:]
    cid = lax.axis_index("c")
    sid = lax.axis_index("s")
    w = sid * 2 + cid
    base = w * _EW
    pltpu.sync_copy(rden_hbm, rden_v)

    zero = jnp.zeros((16,), jnp.float32)
    for j in range(16):
        for c in range(8):
            rows_v[j, pl.ds(c * 16, 16)] = zero

    def zstep(i, carry):
        pltpu.sync_copy(rows_v.at[pl.ds(0, 16)],
                        shared.at[pl.ds(sid * 640 + i * 16, 16)])
        return carry

    lax.fori_loop(0, 40, zstep, 0)
    plsc.subcore_barrier()

    def _didx(g):
        return dst_v[pl.ds(g * 16, 16)]

    def _fire_gather(b, g):
        pltpu.async_copy(h_hbm.at[src_v[pl.ds(g * 16, 16)]],
                         rows_v.at[pl.ds(b * 16, 16)], gsem[b])

    def _wait_gather(b):
        pltpu.make_async_copy(h_hbm.at[src_v[pl.ds(0, 16)]],
                              rows_v.at[pl.ds(b * 16, 16)], gsem[b]).wait()

    def _fire_scatter(b, g):
        pltpu.async_copy(rows_v.at[pl.ds(b * 16, 16)], shared.at[_didx(g)],
                         ssem[b], add=True)

    def _wait_scatter(b):
        pltpu.make_async_copy(rows_v.at[pl.ds(b * 16, 16)],
                              shared.at[_didx(0)], ssem[b]).wait()

    def _compute(b, g):
        att = ex_v[pl.ds(g * 16, 16)] * plsc.load_gather(rden_v, [_didx(g)])
        for j in range(16):
            a = att[j]
            r = b * 16 + j
            for c in range(8):
                s2 = (r, pl.ds(c * 16, 16))
                rows_v[s2] = rows_v[s2] * a

    def chunk(ch, carry):
        cbase = base + ch * _EC
        pltpu.sync_copy(ex_hbm.at[pl.ds(cbase, _EC)], ex_v)
        pltpu.sync_copy(src_hbm.at[pl.ds(cbase, _EC)], src_v)
        pltpu.sync_copy(dst_hbm.at[pl.ds(cbase, _EC)], dst_v)
        for b in range(5):
            _fire_gather(b, b)

        def block(outer, carry2):
            g0 = outer * 10
            for b in range(10):
                g = g0 + b
                bp = (b + 5) % 10

                @pl.when(g >= 5)
                def _():
                    _wait_scatter(bp)

                _fire_gather(bp, g + 5)
                _wait_gather(b)
                _compute(b, g)
                _fire_scatter(b, g)
            return carry2

        lax.fori_loop(0, (_EC // 16 - 5) // 10, block, 0)
        for b in range(5):
            g = _EC // 16 - 5 + b
            _wait_gather(b)
            _compute(b, g)
            _fire_scatter(b, g)
        for b in range(10):
            _wait_scatter(b)
        return carry

    lax.fori_loop(0, _EW // _EC, chunk, 0)
    plsc.subcore_barrier()
    sl = pl.ds(sid * 640, 640)
    pltpu.sync_copy(shared.at[sl], outp_hbm.at[cid, sl])


def _sc_e4(h, ex, rden, src, dst):
    return pl.kernel(
        _e4_body,
        out_type=jax.ShapeDtypeStruct((2, _NP, _F), jnp.float32),
        mesh=_MESH,
        compiler_params=_SC_PARAMS,
        scratch_types=[
            pltpu.VMEM((_NP,), jnp.float32),
            pltpu.VMEM((_EC,), jnp.float32),
            pltpu.VMEM((_EC,), jnp.int32),
            pltpu.VMEM((_EC,), jnp.int32),
            pltpu.VMEM((160, _F), jnp.float32),
            pltpu.VMEM_SHARED((_NP, _F), jnp.float32),
        ] + [pltpu.SemaphoreType.DMA] * 20,
    )(h, ex, rden, src, dst)


_ECH = _E // 16
_ESUB = 4000


def _f1_body(sup_hbm, src_hbm, dst_hbm, aggp_hbm, degp_hbm,
             sup_v, src_v, dst_v, agg_v, deg_v, sup_sh):
    cid = lax.axis_index("c")
    sid = lax.axis_index("s")
    ebase = sid * _ECH
    c0 = cid * 32

    @pl.when(sid == 0)
    def _():
        pltpu.sync_copy(sup_hbm, sup_sh)

    plsc.subcore_barrier()
    pltpu.sync_copy(sup_sh, sup_v)
    zero = jnp.zeros((16,), jnp.float32)

    def z1(i, carry):
        agg_v[pl.ds(i * 16, 16)] = zero
        return carry

    lax.fori_loop(0, (_V * 32) // 16, z1, 0)

    def z2(i, carry):
        deg_v[pl.ds(i * 16, 16)] = zero
        return carry

    lax.fori_loop(0, _V // 16, z2, 0)

    def sub(s, carry):
        pltpu.sync_copy(src_hbm.at[pl.ds(ebase + s * _ESUB, _ESUB)], src_v)
        pltpu.sync_copy(dst_hbm.at[pl.ds(ebase + s * _ESUB, _ESUB)], dst_v)

        def step(i, carry2):
            sl = pl.ds(i * 16, 16)
            sidx = src_v[sl]
            didx = dst_v[sl]
            mask = (sidx < _V) & (didx < _V)
            nhit = plsc.all_reduce_population_count(mask)

            @pl.when(nhit[0] > 0)
            def _():
                sidx_s = jnp.where(mask, sidx, 0)
                didx_s = jnp.where(mask, didx, 0)
                sflat = sidx_s * 64 + c0
                dflat = didx_s * 32
                for c in range(32):
                    v = plsc.load_gather(sup_v, [sflat + c])
                    plsc.addupdate_scatter(agg_v, [dflat + c], v, mask=mask)
                plsc.addupdate_scatter(
                    deg_v, [didx_s], jnp.where(mask, 1.0, 0.0))

            return carry2

        lax.fori_loop(0, _ESUB // 16, step, 0)
        return carry

    lax.fori_loop(0, _ECH // _ESUB, sub, 0)
    w = sid * 2 + cid
    pltpu.sync_copy(agg_v, aggp_hbm.at[pl.ds(w * _V * 32, _V * 32)])

    @pl.when(cid == 0)
    def _():
        pltpu.sync_copy(deg_v, degp_hbm.at[pl.ds(sid * _V, _V)])


def _sc_f1(sup_flat, src, dst):
    return pl.kernel(
        _f1_body,
        out_type=(jax.ShapeDtypeStruct((_NW * _V * 32,), jnp.float32),
                  jax.ShapeDtypeStruct((16 * _V,), jnp.float32)),
        mesh=_MESH,
        compiler_params=_SC_PARAMS,
        scratch_types=[
            pltpu.VMEM((_V * 64,), jnp.float32),
            pltpu.VMEM((_ESUB,), jnp.int32),
            pltpu.VMEM((_ESUB,), jnp.int32),
            pltpu.VMEM((_V * 32,), jnp.float32),
            pltpu.VMEM((_V,), jnp.float32),
            pltpu.VMEM_SHARED((_V * 64,), jnp.float32),
        ],
    )(sup_flat, src, dst)


def _gat_layer_sc(h, ls, ld, src, dst):
    e, mx = _sc_e1(ls, ld, src, dst)
    ex, denp = _sc_e2(e, dst, mx)
    rden = _sc_e3(denp)
    return _sc_e4(h, ex, rden, src, dst)


def kernel(x, adj, W_emb, b_emb, Wg1, a1s, a1d, Wg2, a2s, a2d, Wc, bc):
    src = adj[0].astype(jnp.int32)
    dst = adj[1].astype(jnp.int32)

    h1p, ls1, ld1 = _dense1(x, W_emb, b_emb, Wg1, a1s, a1d)
    p1 = _gat_layer_sc(h1p, ls1[:, 0], ld1[:, 0], src, dst)

    h2p, ls2, ld2 = _dense2(p1[0, :_N], p1[1, :_N], Wg2, a2s, a2d)
    p2 = _gat_layer_sc(h2p, ls2[:, 0], ld2[:, 0], src, dst)

    sup = _support_tc(p2[0, :_V], p2[1, :_V], Wc)
    aggp, degp = _sc_f1(sup.reshape(-1), src, dst)
    return _finalize_tc(sup, aggp.reshape(16, 2, _V, 32),
                        degp.reshape(16, _V), bc)
